# speedup vs baseline: 3.0052x; 3.0052x over previous
import functools

import jax
import jax.numpy as jnp
from jax import lax
from jax.experimental import pallas as pl
from jax.experimental.pallas import tpu as pltpu
from jax.experimental.pallas import tpu_sc as plsc

B = 16384
T = 4
D = 256
NC, NS = 2, 16
NW = NC * NS
BPW = B // NW
C = 32
NCHUNK = BPW // C
IPC = C * T


def _sc_body(idx_hbm, table_hbm, out_hbm, idx_v, rows_v, out_v, sem):
    wid = lax.axis_index("s") * NC + lax.axis_index("c")
    base = wid * BPW
    pltpu.sync_copy(idx_hbm.at[pl.ds(base * T, BPW * T)], idx_v)

    def chunk_body(c, carry):
        pltpu.async_copy(
            table_hbm.at[idx_v.at[pl.ds(c * IPC, IPC)]], rows_v, sem
        ).wait()

        def row_body(r, rcarry):
            for d in range(D // 16):
                s = pl.ds(d * 16, 16)
                acc = rows_v[4 * r, s] + rows_v[4 * r + 1, s]
                acc = acc + rows_v[4 * r + 2, s]
                acc = acc + rows_v[4 * r + 3, s]
                out_v[r, s] = acc
            return rcarry

        lax.fori_loop(0, C, row_body, 0, unroll=False)
        pltpu.sync_copy(out_v, out_hbm.at[pl.ds(base + c * C, C)])
        return carry

    lax.fori_loop(0, NCHUNK, chunk_body, 0, unroll=False)


_sc_embed = functools.partial(
    pl.kernel,
    out_type=jax.ShapeDtypeStruct((B, D), jnp.float32),
    mesh=plsc.VectorSubcoreMesh(core_axis_name="c", subcore_axis_name="s"),
    scratch_types=[
        pltpu.VMEM((BPW * T,), jnp.int32),
        pltpu.VMEM((IPC, D), jnp.float32),
        pltpu.VMEM((C, D), jnp.float32),
        pltpu.SemaphoreType.DMA,
    ],
)(_sc_body)


def kernel(tokens_batch_indices, embedding_weight):
    idx_flat = tokens_batch_indices.astype(jnp.int32).reshape(-1)
    return _sc_embed(idx_flat, embedding_weight)


# speedup vs baseline: 3.7461x; 1.2465x over previous
import functools

import jax
import jax.numpy as jnp
from jax import lax
from jax.experimental import pallas as pl
from jax.experimental.pallas import tpu as pltpu
from jax.experimental.pallas import tpu_sc as plsc

B = 16384
T = 4
D = 256
NC, NS = 2, 16
NW = NC * NS
BPW = B // NW
C = 32
NCHUNK = BPW // C
IPC = C * T


def _sc_body(idx_hbm, table_hbm, out_hbm,
             idx_v, rows0, rows1, out0, out1,
             sem_g0, sem_g1, sem_o0, sem_o1):
    wid = lax.axis_index("s") * NC + lax.axis_index("c")
    base = wid * BPW
    pltpu.sync_copy(idx_hbm.at[pl.ds(base * T, BPW * T)], idx_v)

    rows = (rows0, rows1)
    outs = (out0, out1)
    sem_g = (sem_g0, sem_g1)
    sem_o = (sem_o0, sem_o1)

    def gsrc(c):
        return table_hbm.at[idx_v.at[pl.ds(c * IPC, IPC)]]

    def odst(c):
        return out_hbm.at[pl.ds(base + c * C, C)]

    pltpu.async_copy(gsrc(0), rows0, sem_g0)
    pltpu.async_copy(gsrc(1), rows1, sem_g1)

    for c in range(NCHUNK):
        p = c % 2
        rb, ob = rows[p], outs[p]
        pltpu.make_async_copy(gsrc(c), rb, sem_g[p]).wait()
        if c >= 2:
            pltpu.make_async_copy(outs[p], odst(c - 2), sem_o[p]).wait()

        def row_body(r, carry, rb=rb, ob=ob):
            for d in range(D // 16):
                s = pl.ds(d * 16, 16)
                acc = rb[4 * r, s] + rb[4 * r + 1, s]
                acc = acc + rb[4 * r + 2, s]
                acc = acc + rb[4 * r + 3, s]
                ob[r, s] = acc
            return carry

        lax.fori_loop(0, C, row_body, 0, unroll=False)
        if c + 2 < NCHUNK:
            pltpu.async_copy(gsrc(c + 2), rb, sem_g[p])
        pltpu.async_copy(ob, odst(c), sem_o[p])

    for c in (NCHUNK - 2, NCHUNK - 1):
        pltpu.make_async_copy(outs[c % 2], odst(c), sem_o[c % 2]).wait()


_sc_embed = functools.partial(
    pl.kernel,
    out_type=jax.ShapeDtypeStruct((B, D), jnp.float32),
    mesh=plsc.VectorSubcoreMesh(core_axis_name="c", subcore_axis_name="s"),
    scratch_types=[
        pltpu.VMEM((BPW * T,), jnp.int32),
        pltpu.VMEM((IPC, D), jnp.float32),
        pltpu.VMEM((IPC, D), jnp.float32),
        pltpu.VMEM((C, D), jnp.float32),
        pltpu.VMEM((C, D), jnp.float32),
        pltpu.SemaphoreType.DMA,
        pltpu.SemaphoreType.DMA,
        pltpu.SemaphoreType.DMA,
        pltpu.SemaphoreType.DMA,
    ],
)(_sc_body)


def kernel(tokens_batch_indices, embedding_weight):
    idx_flat = tokens_batch_indices.astype(jnp.int32).reshape(-1)
    return _sc_embed(idx_flat, embedding_weight)


# speedup vs baseline: 4.0578x; 1.0832x over previous
import functools

import jax
import jax.numpy as jnp
from jax import lax
from jax.experimental import pallas as pl
from jax.experimental.pallas import tpu as pltpu
from jax.experimental.pallas import tpu_sc as plsc

B = 16384
T = 4
D = 256
NC, NS = 2, 16
NW = NC * NS
BPW = B // NW
C = 32
NCHUNK = BPW // C
IPC = C * T


def _sc_body(idx_hbm, table_hbm, out_hbm,
             idx_v, rows0, rows1, out0, out1,
             sem_g0, sem_g1, sem_o0, sem_o1):
    wid = lax.axis_index("s") * NC + lax.axis_index("c")
    base = wid * BPW
    pltpu.sync_copy(idx_hbm.at[pl.ds(base * T, BPW * T)], idx_v)

    rows = (rows0, rows1)
    outs = (out0, out1)
    sem_g = (sem_g0, sem_g1)
    sem_o = (sem_o0, sem_o1)

    def gsrc(c):
        return table_hbm.at[idx_v.at[pl.ds(c * IPC, IPC)]]

    def odst(c):
        return out_hbm.at[pl.ds(base + c * C, C)]

    pltpu.async_copy(gsrc(0), rows0, sem_g0)
    pltpu.async_copy(gsrc(1), rows1, sem_g1)

    def pair_body(p, carry):
        for b in range(2):
            c = 2 * p + b
            rb, ob = rows[b], outs[b]
            pltpu.make_async_copy(gsrc(c), rb, sem_g[b]).wait()

            @pl.when(p >= 1)
            def _drain(ob=ob, c=c, b=b):
                pltpu.make_async_copy(ob, odst(c - 2), sem_o[b]).wait()

            @plsc.parallel_loop(0, C, unroll=4)
            def row_body(r, rb=rb, ob=ob):
                for d in range(D // 16):
                    s = pl.ds(d * 16, 16)
                    acc = rb[4 * r, s] + rb[4 * r + 1, s]
                    acc = acc + rb[4 * r + 2, s]
                    acc = acc + rb[4 * r + 3, s]
                    ob[r, s] = acc

            @pl.when(p < NCHUNK // 2 - 1)
            def _next(rb=rb, c=c, b=b):
                pltpu.async_copy(gsrc(c + 2), rb, sem_g[b])

            pltpu.async_copy(ob, odst(c), sem_o[b])
        return carry

    lax.fori_loop(0, NCHUNK // 2, pair_body, 0, unroll=False)

    for c in (NCHUNK - 2, NCHUNK - 1):
        pltpu.make_async_copy(outs[c % 2], odst(c), sem_o[c % 2]).wait()


_sc_embed = functools.partial(
    pl.kernel,
    out_type=jax.ShapeDtypeStruct((B, D), jnp.float32),
    mesh=plsc.VectorSubcoreMesh(core_axis_name="c", subcore_axis_name="s"),
    scratch_types=[
        pltpu.VMEM((BPW * T,), jnp.int32),
        pltpu.VMEM((IPC, D), jnp.float32),
        pltpu.VMEM((IPC, D), jnp.float32),
        pltpu.VMEM((C, D), jnp.float32),
        pltpu.VMEM((C, D), jnp.float32),
        pltpu.SemaphoreType.DMA,
        pltpu.SemaphoreType.DMA,
        pltpu.SemaphoreType.DMA,
        pltpu.SemaphoreType.DMA,
    ],
)(_sc_body)


def kernel(tokens_batch_indices, embedding_weight):
    idx_flat = tokens_batch_indices.astype(jnp.int32).reshape(-1)
    return _sc_embed(idx_flat, embedding_weight)


# speedup vs baseline: 5.3618x; 1.3214x over previous
import functools

import jax
import jax.numpy as jnp
from jax import lax
from jax.experimental import pallas as pl
from jax.experimental.pallas import tpu as pltpu
from jax.experimental.pallas import tpu_sc as plsc

B = 16384
T = 4
D = 256
V = 304
NC, NS = 2, 16
NW = NC * NS
BPW = B // NW
C = 64
NCHUNK = BPW // C


def _sc_body(idx_hbm, table_hbm, out_hbm,
             idx_v, table_v, out0, out1, sem_o0, sem_o1):
    wid = lax.axis_index("s") * NC + lax.axis_index("c")
    base = wid * BPW
    pltpu.sync_copy(table_hbm, table_v)
    pltpu.sync_copy(idx_hbm.at[pl.ds(base * T, BPW * T)], idx_v)

    outs = (out0, out1)
    sem_o = (sem_o0, sem_o1)

    def odst(c):
        return out_hbm.at[pl.ds(base + c * C, C)]

    iota = lax.iota(jnp.int32, 16)

    def pair_body(p, carry):
        for bu in range(2):
            c = 2 * p + bu
            ob = outs[bu]

            @pl.when(p >= 1)
            def _drain(ob=ob, c=c, bu=bu):
                pltpu.make_async_copy(ob, odst(c - 2), sem_o[bu]).wait()

            @plsc.parallel_loop(0, C, unroll=2)
            def row_body(r, c=c, ob=ob):
                off = (c * C + r) * T
                offv = jnp.full((16,), off, dtype=jnp.int32)
                sidx = []
                for t in range(T):
                    tok = plsc.load_gather(idx_v, [offv + t])
                    sidx.append((tok << 8) + iota)
                for d in range(D // 16):
                    dof = 16 * d
                    a = plsc.load_gather(table_v, [sidx[0] + dof])
                    b2 = plsc.load_gather(table_v, [sidx[1] + dof])
                    c2 = plsc.load_gather(table_v, [sidx[2] + dof])
                    d2 = plsc.load_gather(table_v, [sidx[3] + dof])
                    ob[r, pl.ds(dof, 16)] = (a + b2) + (c2 + d2)

            pltpu.async_copy(ob, odst(c), sem_o[bu])
        return carry

    lax.fori_loop(0, NCHUNK // 2, pair_body, 0, unroll=False)

    for c in (NCHUNK - 2, NCHUNK - 1):
        pltpu.make_async_copy(outs[c % 2], odst(c), sem_o[c % 2]).wait()


_sc_embed = functools.partial(
    pl.kernel,
    out_type=jax.ShapeDtypeStruct((B, D), jnp.float32),
    mesh=plsc.VectorSubcoreMesh(core_axis_name="c", subcore_axis_name="s"),
    compiler_params=pltpu.CompilerParams(needs_layout_passes=False),
    scratch_types=[
        pltpu.VMEM((BPW * T,), jnp.int32),
        pltpu.VMEM((V * D,), jnp.float32),
        pltpu.VMEM((C, D), jnp.float32),
        pltpu.VMEM((C, D), jnp.float32),
        pltpu.SemaphoreType.DMA,
        pltpu.SemaphoreType.DMA,
    ],
)(_sc_body)


def kernel(tokens_batch_indices, embedding_weight):
    idx_flat = tokens_batch_indices.astype(jnp.int32).reshape(-1)
    return _sc_embed(idx_flat, embedding_weight.reshape(-1))


# speedup vs baseline: 6.6433x; 1.2390x over previous
import functools

import jax
import jax.numpy as jnp
from jax import lax
from jax.experimental import pallas as pl
from jax.experimental.pallas import tpu as pltpu
from jax.experimental.pallas import tpu_sc as plsc

B = 16384
T = 4
D = 256
V = 304
NC, NS = 2, 16
NW = NC * NS
BPW = B // NW
C = 64
NCHUNK = BPW // C


def _sc_body(idx_hbm, table_hbm, out_hbm,
             idx_v, table_v, out0, out1, sem_o0, sem_o1):
    wid = lax.axis_index("s") * NC + lax.axis_index("c")
    base = wid * BPW
    pltpu.sync_copy(table_hbm, table_v)
    pltpu.sync_copy(idx_hbm.at[pl.ds(base * T, BPW * T)], idx_v)

    outs = (out0, out1)
    sem_o = (sem_o0, sem_o1)

    def odst(c):
        return out_hbm.at[pl.ds(base + c * C, C)]

    iota = lax.iota(jnp.int32, 16)

    def pair_body(p, carry):
        for bu in range(2):
            c = 2 * p + bu
            ob = outs[bu]

            @pl.when(p >= 1)
            def _drain(ob=ob, c=c, bu=bu):
                pltpu.make_async_copy(ob, odst(c - 2), sem_o[bu]).wait()

            @plsc.parallel_loop(0, C, unroll=2)
            def row_body(r, c=c, ob=ob):
                off = (c * C + r) * T
                offv = jnp.full((16,), off, dtype=jnp.int32)
                sidx = []
                for t in range(T):
                    tok = plsc.load_gather(idx_v, [offv + t])
                    sidx.append((tok << 8) + iota)
                gl = V * D - 16 * (D // 16 - 1)
                for d in range(D // 16):
                    dof = 16 * d
                    tv = table_v.at[pl.ds(dof, gl)]
                    a = plsc.load_gather(tv, [sidx[0]])
                    b2 = plsc.load_gather(tv, [sidx[1]])
                    c2 = plsc.load_gather(tv, [sidx[2]])
                    d2 = plsc.load_gather(tv, [sidx[3]])
                    ob[r, pl.ds(dof, 16)] = (a + b2) + (c2 + d2)

            pltpu.async_copy(ob, odst(c), sem_o[bu])
        return carry

    lax.fori_loop(0, NCHUNK // 2, pair_body, 0, unroll=False)

    for c in (NCHUNK - 2, NCHUNK - 1):
        pltpu.make_async_copy(outs[c % 2], odst(c), sem_o[c % 2]).wait()


_sc_embed = functools.partial(
    pl.kernel,
    out_type=jax.ShapeDtypeStruct((B, D), jnp.float32),
    mesh=plsc.VectorSubcoreMesh(core_axis_name="c", subcore_axis_name="s"),
    compiler_params=pltpu.CompilerParams(needs_layout_passes=False),
    scratch_types=[
        pltpu.VMEM((BPW * T,), jnp.int32),
        pltpu.VMEM((V * D,), jnp.float32),
        pltpu.VMEM((C, D), jnp.float32),
        pltpu.VMEM((C, D), jnp.float32),
        pltpu.SemaphoreType.DMA,
        pltpu.SemaphoreType.DMA,
    ],
)(_sc_body)


def kernel(tokens_batch_indices, embedding_weight):
    idx_flat = tokens_batch_indices.astype(jnp.int32).reshape(-1)
    return _sc_embed(idx_flat, embedding_weight.reshape(-1))


# speedup vs baseline: 6.6550x; 1.0018x over previous
import functools

import jax
import jax.numpy as jnp
from jax import lax
from jax.experimental import pallas as pl
from jax.experimental.pallas import tpu as pltpu
from jax.experimental.pallas import tpu_sc as plsc

B = 16384
T = 4
D = 256
V = 304
NC, NS = 2, 16
NW = NC * NS
BPW = B // NW
C = 64
NCHUNK = BPW // C


def _sc_body(idx_hbm, table_hbm, out_hbm,
             idx_v, table_v, out0, out1, sem_o0, sem_o1):
    wid = lax.axis_index("s") * NC + lax.axis_index("c")
    base = wid * BPW
    pltpu.sync_copy(table_hbm, table_v)
    pltpu.sync_copy(idx_hbm.at[pl.ds(base * T, BPW * T)], idx_v)

    outs = (out0, out1)
    sem_o = (sem_o0, sem_o1)

    def odst(c):
        return out_hbm.at[pl.ds(base + c * C, C)]

    iota = lax.iota(jnp.int32, 16)

    def pair_body(p, carry):
        for bu in range(2):
            c = 2 * p + bu
            ob = outs[bu]

            @pl.when(p >= 1)
            def _drain(ob=ob, c=c, bu=bu):
                pltpu.make_async_copy(ob, odst(c - 2), sem_o[bu]).wait()

            @plsc.parallel_loop(0, C, unroll=2)
            def row_body(r, c=c, ob=ob):
                off = (c * C + r) * T
                offv = jnp.full((16,), off, dtype=jnp.int32)
                sidx = []
                for t in range(T):
                    tok = plsc.load_gather(idx_v, [offv + t])
                    sidx.append((tok << 8) + iota)
                gl = V * D - 16 * (D // 16 - 1)
                for d in range(D // 16):
                    dof = 16 * d
                    tv = table_v.at[pl.ds(dof, gl)]
                    a = plsc.load_gather(tv, [sidx[0]])
                    b2 = plsc.load_gather(tv, [sidx[1]])
                    c2 = plsc.load_gather(tv, [sidx[2]])
                    d2 = plsc.load_gather(tv, [sidx[3]])
                    ob[r, pl.ds(dof, 16)] = (a + b2) + (c2 + d2)

            pltpu.async_copy(ob, odst(c), sem_o[bu])
        return carry

    lax.fori_loop(0, NCHUNK // 2, pair_body, 0, unroll=False)

    for c in (NCHUNK - 2, NCHUNK - 1):
        pltpu.make_async_copy(outs[c % 2], odst(c), sem_o[c % 2]).wait()


_sc_embed = functools.partial(
    pl.kernel,
    out_type=jax.ShapeDtypeStruct((B, D), jnp.float32),
    mesh=plsc.VectorSubcoreMesh(core_axis_name="c", subcore_axis_name="s"),
    compiler_params=pltpu.CompilerParams(
        needs_layout_passes=False,
        skip_device_barrier=True,
        disable_bounds_checks=True,
        disable_semaphore_checks=True,
    ),
    scratch_types=[
        pltpu.VMEM((BPW * T,), jnp.int32),
        pltpu.VMEM((V * D,), jnp.float32),
        pltpu.VMEM((C, D), jnp.float32),
        pltpu.VMEM((C, D), jnp.float32),
        pltpu.SemaphoreType.DMA,
        pltpu.SemaphoreType.DMA,
    ],
)(_sc_body)


def kernel(tokens_batch_indices, embedding_weight):
    idx_flat = tokens_batch_indices.astype(jnp.int32).reshape(-1)
    return _sc_embed(idx_flat, embedding_weight.reshape(-1))
